# baseline (speedup 1.0000x reference)
import jax
import jax.numpy as jnp
from jax import lax
from jax.experimental import pallas as pl
from jax.experimental.pallas import tpu as pltpu
from jax.experimental.pallas import tpu_sc as plsc

N_NODES = 10000
D_IN = 128
D_OUT = 64
DR = 128

NC = 2
NS = 16
NW = NC * NS

CHUNK = 128
ACC_PAD = 10240
TPW = ACC_PAD // NS

_mesh = plsc.VectorSubcoreMesh(
    core_axis_name="c", subcore_axis_name="s", num_cores=NC, num_subcores=NS)


def _worker_id():
    return lax.axis_index("s") * NC + lax.axis_index("c")


def _deg_body(dst2, deg_out, idx_v, ones_v, zb_v, acc_sh, sem):
    del sem
    c = lax.axis_index("c")
    s = lax.axis_index("s")
    w = _worker_id()

    def fo(i, _):
        ones_v[pl.ds(i * 16, 16)] = jnp.ones((16,), jnp.float32)
        return 0
    lax.fori_loop(0, CHUNK // 16, fo, 0)

    def fz(i, _):
        zb_v[pl.ds(i * 16, 16)] = jnp.zeros((16,), jnp.float32)
        return 0
    lax.fori_loop(0, TPW // 16, fz, 0)

    pltpu.sync_copy(zb_v, acc_sh.at[pl.ds(s * TPW, TPW)])
    pltpu.sync_copy(dst2.at[w], idx_v)
    plsc.subcore_barrier()

    def body(j, _):
        pltpu.sync_copy(ones_v, acc_sh.at[idx_v.at[j]], add=True)
        return 0
    lax.fori_loop(0, idx_v.shape[0], body, 0)

    plsc.subcore_barrier()
    pltpu.sync_copy(acc_sh.at[pl.ds(s * TPW, TPW)], zb_v)
    pltpu.sync_copy(zb_v, deg_out.at[pl.ds(c * ACC_PAD + s * TPW, TPW)])


def _deg_call(dst2):
    nch = dst2.shape[1]
    k = pl.kernel(
        _deg_body,
        out_type=jax.ShapeDtypeStruct((NC * ACC_PAD,), jnp.float32),
        mesh=_mesh,
        scratch_types=[
            pltpu.VMEM((nch, CHUNK), jnp.int32),
            pltpu.VMEM((CHUNK,), jnp.float32),
            pltpu.VMEM((TPW,), jnp.float32),
            pltpu.VMEM_SHARED((ACC_PAD,), jnp.float32),
            pltpu.SemaphoreType.DMA,
        ],
    )
    return k(dst2)


NBLK = 2


def _agg_body(rows_hbm, src2, dst2, zer_hbm, out_hbm,
              idx_s, idx_d, rows_v, acc_sh, sg0, sg1):
    c = lax.axis_index("c")
    s = lax.axis_index("s")
    w = _worker_id()
    nbc = idx_s.shape[0]
    b0 = rows_v.at[0]
    b1 = rows_v.at[1]

    def zr(j, _):
        off = s * TPW + j * CHUNK
        pltpu.sync_copy(zer_hbm.at[pl.ds(j * CHUNK, CHUNK)], b0)
        pltpu.sync_copy(b0, acc_sh.at[pl.ds(off, CHUNK)])
        return 0
    lax.fori_loop(0, TPW // CHUNK, zr, 0)
    plsc.subcore_barrier()

    def blk(b, _):
        pltpu.sync_copy(src2.at[w, pl.ds(b * nbc, nbc)], idx_s)
        pltpu.sync_copy(dst2.at[w, pl.ds(b * nbc, nbc)], idx_d)

        def body(j, _):
            pltpu.async_copy(rows_hbm.at[idx_s.at[j]], b0, sg0).wait()
            pltpu.sync_copy(b0, acc_sh.at[idx_d.at[j]], add=True)
            return 0
        lax.fori_loop(0, nbc, body, 0)
        return 0
    lax.fori_loop(0, NBLK, blk, 0)

    plsc.subcore_barrier()

    def dmp(j, _):
        off = s * TPW + j * CHUNK
        pltpu.sync_copy(acc_sh.at[pl.ds(off, CHUNK)], b0)
        pltpu.sync_copy(b0, out_hbm.at[c, pl.ds(off, CHUNK)])
        return 0
    lax.fori_loop(0, TPW // CHUNK, dmp, 0)


def _agg_call(rows, src2, dst2, zer_c):
    nch = src2.shape[1]
    k = pl.kernel(
        _agg_body,
        out_type=jax.ShapeDtypeStruct((NC, ACC_PAD, DR), jnp.float32),
        mesh=_mesh,
        scratch_types=[
            pltpu.VMEM((nch // NBLK, CHUNK), jnp.int32),
            pltpu.VMEM((nch // NBLK, CHUNK), jnp.int32),
            pltpu.VMEM((2, CHUNK, DR), jnp.float32),
            pltpu.VMEM_SHARED((ACC_PAD, DR), jnp.float32),
            pltpu.SemaphoreType.DMA,
            pltpu.SemaphoreType.DMA,
        ],
    )
    return k(rows, src2, dst2, zer_c)


def _dense1_body(degp, x, w1, h1s_o, dinv_o):
    deg = degp[0] + degp[1] + 1.0
    dinv = lax.rsqrt(deg)[:, None]
    h1 = jnp.dot(x[...], w1[...], preferred_element_type=jnp.float32)
    h1s = h1 * dinv
    h1s_o[...] = jnp.concatenate([h1s, jnp.zeros_like(h1s)], axis=1)
    dinv_o[...] = dinv


def _dense2_body(t1p, dinv, h1s, b1, gamma, beta, w2, w3, b3, h2s_o, out2_o):
    dv = dinv[...]
    conv1 = (t1p[0, :, :D_OUT] + t1p[1, :, :D_OUT]
             + h1s[:, :D_OUT]) * dv + b1[...]
    mu = jnp.mean(conv1, axis=-1, keepdims=True)
    d = conv1 - mu
    var = jnp.mean(d * d, axis=-1, keepdims=True)
    h = jnp.maximum(d * lax.rsqrt(var + 1e-5) * gamma[...] + beta[...], 0.0)
    g2 = jnp.dot(h, w2[...], preferred_element_type=jnp.float32)
    h2s = g2 * dv
    h2s_o[...] = jnp.concatenate([h2s, jnp.zeros_like(h2s)], axis=1)
    out2_o[...] = jax.nn.sigmoid(
        jnp.dot(h, w3[...], preferred_element_type=jnp.float32) + b3[...])


def _dense3_body(t2p, dinv, h2s, b2, out1_o):
    out1_o[...] = (t2p[0, :, :D_OUT] + t2p[1, :, :D_OUT]
                   + h2s[:, :D_OUT]) * dinv[...] + b2[...]


def _tc_call(body, out_shapes, *args):
    return pl.pallas_call(body, out_shape=out_shapes)(*args)


@jax.jit
def kernel(x, edge_index, W1, b1, gamma, beta, W2, b2, W3, b3):
    n = x.shape[0]
    e = edge_index.shape[1]
    q = NW * CHUNK * 2 * 8
    epw = ((e + q - 1) // q) * CHUNK * 2 * 8
    tot = epw * NW
    pad = tot - e

    src = edge_index[0]
    dst = edge_index[1]
    src_p = jnp.concatenate([src, jnp.zeros((pad,), jnp.int32)])
    dst_p = jnp.concatenate(
        [dst, n + (jnp.arange(pad, dtype=jnp.int32) % (ACC_PAD - n))])
    src2 = src_p.reshape(NW, epw // CHUNK, CHUNK)
    dst2 = dst_p.reshape(NW, epw // CHUNK, CHUNK)

    zer_c = jnp.zeros((TPW, DR), jnp.float32)

    b1r = b1.reshape(1, D_OUT)
    b2r = b2.reshape(1, D_OUT)
    b3r = b3.reshape(1, b3.shape[0])
    gr = gamma.reshape(1, D_OUT)
    br = beta.reshape(1, D_OUT)

    deg_p = _deg_call(dst2).reshape(NC, ACC_PAD)
    x_pad = jnp.pad(x, ((0, ACC_PAD - n), (0, 0)))

    h1s, dinv = _tc_call(
        _dense1_body,
        (jax.ShapeDtypeStruct((ACC_PAD, DR), jnp.float32),
         jax.ShapeDtypeStruct((ACC_PAD, 1), jnp.float32)),
        deg_p, x_pad, W1)

    t1_p = _agg_call(h1s, src2, dst2, zer_c)

    h2s, out2 = _tc_call(
        _dense2_body,
        (jax.ShapeDtypeStruct((ACC_PAD, DR), jnp.float32),
         jax.ShapeDtypeStruct((ACC_PAD, W3.shape[1]), jnp.float32)),
        t1_p, dinv, h1s, b1r, gr, br, W2, W3, b3r)

    t2_p = _agg_call(h2s, src2, dst2, zer_c)

    out1 = _tc_call(
        _dense3_body,
        jax.ShapeDtypeStruct((ACC_PAD, D_OUT), jnp.float32),
        t2_p, dinv, h2s, b2r)

    return (out1[:n], out2[:n])


# speedup vs baseline: 3.3469x; 3.3469x over previous
import jax
import jax.numpy as jnp
from jax import lax
from jax.experimental import pallas as pl
from jax.experimental.pallas import tpu as pltpu
from jax.experimental.pallas import tpu_sc as plsc

N_NODES = 10000
D_IN = 128
D_OUT = 64
DR = 128

NC = 2
NS = 16
NW = NC * NS

CHUNK = 128
ACC_PAD = 10240
TPW = ACC_PAD // NS

_mesh = plsc.VectorSubcoreMesh(
    core_axis_name="c", subcore_axis_name="s", num_cores=NC, num_subcores=NS)


def _worker_id():
    return lax.axis_index("s") * NC + lax.axis_index("c")


def _deg_body(dst2, deg_out, idx_v, ones_v, zb_v, acc_sh, sem):
    del sem
    c = lax.axis_index("c")
    s = lax.axis_index("s")
    w = _worker_id()

    def fo(i, _):
        ones_v[pl.ds(i * 16, 16)] = jnp.ones((16,), jnp.float32)
        return 0
    lax.fori_loop(0, CHUNK // 16, fo, 0)

    def fz(i, _):
        zb_v[pl.ds(i * 16, 16)] = jnp.zeros((16,), jnp.float32)
        return 0
    lax.fori_loop(0, TPW // 16, fz, 0)

    pltpu.sync_copy(zb_v, acc_sh.at[pl.ds(s * TPW, TPW)])
    pltpu.sync_copy(dst2.at[w], idx_v)
    plsc.subcore_barrier()

    def body(j, _):
        pltpu.sync_copy(ones_v, acc_sh.at[idx_v.at[j]], add=True)
        return 0
    lax.fori_loop(0, idx_v.shape[0], body, 0)

    plsc.subcore_barrier()
    pltpu.sync_copy(acc_sh.at[pl.ds(s * TPW, TPW)], zb_v)
    pltpu.sync_copy(zb_v, deg_out.at[pl.ds(c * ACC_PAD + s * TPW, TPW)])


def _deg_call(dst2):
    nch = dst2.shape[1]
    k = pl.kernel(
        _deg_body,
        out_type=jax.ShapeDtypeStruct((NC * ACC_PAD,), jnp.float32),
        mesh=_mesh,
        scratch_types=[
            pltpu.VMEM((nch, CHUNK), jnp.int32),
            pltpu.VMEM((CHUNK,), jnp.float32),
            pltpu.VMEM((TPW,), jnp.float32),
            pltpu.VMEM_SHARED((ACC_PAD,), jnp.float32),
            pltpu.SemaphoreType.DMA,
        ],
    )
    return k(dst2)


NBLK = 2


def _agg_body(rows_hbm, src2, dst2, zer_hbm, out_hbm,
              idx_s, idx_d, rows_v, acc_sh, sg0, sg1):
    c = lax.axis_index("c")
    s = lax.axis_index("s")
    w = _worker_id()
    nbc = idx_s.shape[0]
    b0 = rows_v.at[0]
    b1 = rows_v.at[1]

    def zr(j, _):
        off = s * TPW + j * CHUNK
        pltpu.sync_copy(zer_hbm.at[pl.ds(j * CHUNK, CHUNK)], b0)
        pltpu.sync_copy(b0, acc_sh.at[pl.ds(off, CHUNK)])
        return 0
    lax.fori_loop(0, TPW // CHUNK, zr, 0)
    plsc.subcore_barrier()

    def blk(b, _):
        pltpu.sync_copy(src2.at[w, pl.ds(b * nbc, nbc)], idx_s)
        pltpu.sync_copy(dst2.at[w, pl.ds(b * nbc, nbc)], idx_d)
        pltpu.async_copy(rows_hbm.at[idx_s.at[0]], b0, sg0)

        def pair(jj, _):
            j0 = 2 * jj
            pltpu.async_copy(rows_hbm.at[idx_s.at[j0 + 1]], b1, sg1)
            pltpu.make_async_copy(rows_hbm.at[pl.ds(0, CHUNK)], b0, sg0).wait()
            pltpu.sync_copy(b0, acc_sh.at[idx_d.at[j0]], add=True)

            @pl.when(jj + 1 < nbc // 2)
            def _():
                pltpu.async_copy(rows_hbm.at[idx_s.at[j0 + 2]], b0, sg0)
            pltpu.make_async_copy(rows_hbm.at[pl.ds(0, CHUNK)], b1, sg1).wait()
            pltpu.sync_copy(b1, acc_sh.at[idx_d.at[j0 + 1]], add=True)
            return 0
        lax.fori_loop(0, nbc // 2, pair, 0)
        return 0
    lax.fori_loop(0, NBLK, blk, 0)

    plsc.subcore_barrier()

    def dmp(j, _):
        off = s * TPW + j * CHUNK
        pltpu.sync_copy(acc_sh.at[pl.ds(off, CHUNK)], b0)
        pltpu.sync_copy(b0, out_hbm.at[c, pl.ds(off, CHUNK)])
        return 0
    lax.fori_loop(0, TPW // CHUNK, dmp, 0)


def _agg_call(rows, src2, dst2, zer_c):
    nch = src2.shape[1]
    k = pl.kernel(
        _agg_body,
        out_type=jax.ShapeDtypeStruct((NC, ACC_PAD, DR), jnp.float32),
        mesh=_mesh,
        scratch_types=[
            pltpu.VMEM((nch // NBLK, CHUNK), jnp.int32),
            pltpu.VMEM((nch // NBLK, CHUNK), jnp.int32),
            pltpu.VMEM((2, CHUNK, DR), jnp.float32),
            pltpu.VMEM_SHARED((ACC_PAD, DR), jnp.float32),
            pltpu.SemaphoreType.DMA,
            pltpu.SemaphoreType.DMA,
        ],
    )
    return k(rows, src2, dst2, zer_c)


def _dense1_body(degp, x, w1, h1s_o, dinv_o):
    deg = degp[0] + degp[1] + 1.0
    dinv = lax.rsqrt(deg)[:, None]
    h1 = jnp.dot(x[...], w1[...], preferred_element_type=jnp.float32)
    h1s = h1 * dinv
    h1s_o[...] = jnp.concatenate([h1s, jnp.zeros_like(h1s)], axis=1)
    dinv_o[...] = dinv


def _dense2_body(t1p, dinv, h1s, b1, gamma, beta, w2, w3, b3, h2s_o, out2_o):
    dv = dinv[...]
    conv1 = (t1p[0, :, :D_OUT] + t1p[1, :, :D_OUT]
             + h1s[:, :D_OUT]) * dv + b1[...]
    mu = jnp.mean(conv1, axis=-1, keepdims=True)
    d = conv1 - mu
    var = jnp.mean(d * d, axis=-1, keepdims=True)
    h = jnp.maximum(d * lax.rsqrt(var + 1e-5) * gamma[...] + beta[...], 0.0)
    g2 = jnp.dot(h, w2[...], preferred_element_type=jnp.float32)
    h2s = g2 * dv
    h2s_o[...] = jnp.concatenate([h2s, jnp.zeros_like(h2s)], axis=1)
    out2_o[...] = jax.nn.sigmoid(
        jnp.dot(h, w3[...], preferred_element_type=jnp.float32) + b3[...])


def _dense3_body(t2p, dinv, h2s, b2, out1_o):
    out1_o[...] = (t2p[0, :, :D_OUT] + t2p[1, :, :D_OUT]
                   + h2s[:, :D_OUT]) * dinv[...] + b2[...]


def _tc_call(body, out_shapes, *args):
    return pl.pallas_call(body, out_shape=out_shapes)(*args)


@jax.jit
def kernel(x, edge_index, W1, b1, gamma, beta, W2, b2, W3, b3):
    n = x.shape[0]
    e = edge_index.shape[1]
    q = NW * CHUNK * 2 * 8
    epw = ((e + q - 1) // q) * CHUNK * 2 * 8
    tot = epw * NW
    pad = tot - e

    src = edge_index[0]
    dst = edge_index[1]
    src_p = jnp.concatenate([src, jnp.arange(pad, dtype=jnp.int32) % n])
    dst_p = jnp.concatenate(
        [dst, n + (jnp.arange(pad, dtype=jnp.int32) % (ACC_PAD - n))])
    src2 = src_p.reshape(NW, epw // CHUNK, CHUNK)
    dst2 = dst_p.reshape(NW, epw // CHUNK, CHUNK)

    zer_c = jnp.zeros((TPW, DR), jnp.float32)

    b1r = b1.reshape(1, D_OUT)
    b2r = b2.reshape(1, D_OUT)
    b3r = b3.reshape(1, b3.shape[0])
    gr = gamma.reshape(1, D_OUT)
    br = beta.reshape(1, D_OUT)

    deg_p = _deg_call(dst2).reshape(NC, ACC_PAD)
    x_pad = jnp.pad(x, ((0, ACC_PAD - n), (0, 0)))

    h1s, dinv = _tc_call(
        _dense1_body,
        (jax.ShapeDtypeStruct((ACC_PAD, DR), jnp.float32),
         jax.ShapeDtypeStruct((ACC_PAD, 1), jnp.float32)),
        deg_p, x_pad, W1)

    t1_p = _agg_call(h1s, src2, dst2, zer_c)

    h2s, out2 = _tc_call(
        _dense2_body,
        (jax.ShapeDtypeStruct((ACC_PAD, DR), jnp.float32),
         jax.ShapeDtypeStruct((ACC_PAD, W3.shape[1]), jnp.float32)),
        t1_p, dinv, h1s, b1r, gr, br, W2, W3, b3r)

    t2_p = _agg_call(h2s, src2, dst2, zer_c)

    out1 = _tc_call(
        _dense3_body,
        jax.ShapeDtypeStruct((ACC_PAD, D_OUT), jnp.float32),
        t2_p, dinv, h2s, b2r)

    return (out1[:n], out2[:n])
